# baseline (speedup 1.0000x reference)
import jax
import jax.numpy as jnp
from jax.experimental import pallas as pl
from jax.experimental.pallas import tpu as pltpu


def _linear_kernel(x_ref, w_ref, b_ref, o_ref):
    acc = jax.lax.dot_general(
        x_ref[...], w_ref[...],
        dimension_numbers=(((1,), (1,)), ((), ())),
        preferred_element_type=jnp.float32,
    )
    o_ref[...] = acc + b_ref[...]


def kernel(x, W, b):
    B, K = x.shape
    N = W.shape[0]
    blk = 2048
    b2 = b.reshape(1, N)
    return pl.pallas_call(
        _linear_kernel,
        grid=(B // blk,),
        compiler_params=pltpu.CompilerParams(
            dimension_semantics=("parallel",),
        ),
        in_specs=[
            pl.BlockSpec((blk, K), lambda i: (i, 0)),
            pl.BlockSpec((N, K), lambda i: (0, 0)),
            pl.BlockSpec((1, N), lambda i: (0, 0)),
        ],
        out_specs=pl.BlockSpec((blk, N), lambda i: (i, 0)),
        out_shape=jax.ShapeDtypeStruct((B, N), x.dtype),
    )(x, W, b2)


# speedup vs baseline: 1.0475x; 1.0475x over previous
import jax
import jax.numpy as jnp
from jax.experimental import pallas as pl
from jax.experimental.pallas import tpu as pltpu

NCHUNK = 16
NBUF = 8


def _linear_kernel(x_hbm, w_ref, b_ref, o_ref, xbuf, sems):
    ch = x_hbm.shape[0] // NCHUNK

    def start(i):
        pltpu.make_async_copy(
            x_hbm.at[pl.ds(i * ch, ch), :],
            xbuf.at[i % NBUF],
            sems.at[i % NBUF],
        ).start()

    for i in range(NBUF):
        start(i)
    for i in range(NCHUNK):
        pltpu.make_async_copy(
            x_hbm.at[pl.ds(i * ch, ch), :],
            xbuf.at[i % NBUF],
            sems.at[i % NBUF],
        ).wait()
        acc = jax.lax.dot_general(
            xbuf[i % NBUF], w_ref[...],
            dimension_numbers=(((1,), (1,)), ((), ())),
            preferred_element_type=jnp.float32,
        )
        o_ref[pl.ds(i * ch, ch), :] = acc + b_ref[...]
        if i + NBUF < NCHUNK:
            start(i + NBUF)


def kernel(x, W, b):
    B, K = x.shape
    N = W.shape[0]
    ch = B // NCHUNK
    b2 = b.reshape(1, N)
    return pl.pallas_call(
        _linear_kernel,
        in_specs=[
            pl.BlockSpec(memory_space=pltpu.MemorySpace.HBM),
            pl.BlockSpec((N, K), lambda: (0, 0)),
            pl.BlockSpec((1, N), lambda: (0, 0)),
        ],
        out_specs=pl.BlockSpec((B, N), lambda: (0, 0)),
        out_shape=jax.ShapeDtypeStruct((B, N), x.dtype),
        scratch_shapes=[
            pltpu.VMEM((NBUF, ch, K), jnp.float32),
            pltpu.SemaphoreType.DMA((NBUF,)),
        ],
    )(x, W, b2)
